# Initial kernel scaffold; baseline (speedup 1.0000x reference)
import jax
import jax.numpy as jnp
from jax.experimental import pallas as pl


def kernel(ids, table):
    raise NotImplementedError("write your pallas kernel here")


# speedup vs baseline: 1.1094x; 1.1094x over previous
import functools

import jax
import jax.numpy as jnp
from jax import lax
from jax.experimental import pallas as pl
from jax.experimental.pallas import tpu as pltpu
from jax.experimental.pallas import tpu_sc as plsc

NUM_EMB = 1_000_000
DIM = 32
B_TOTAL = 16384 * 50
NC, NS = 2, 16
NW = NC * NS
B_PER_W = B_TOTAL // NW
G = 128
CHUNK = 2560
GPC = CHUNK // G
NCHUNKS = B_PER_W // CHUNK
assert B_PER_W % CHUNK == 0 and CHUNK % G == 0


def _emb_body(ids_hbm, table_hbm, out_hbm, idx_v, rows_v, sem):
    wid = lax.axis_index("s") * NC + lax.axis_index("c")
    base = wid * B_PER_W

    @pl.loop(0, B_PER_W // CHUNK)
    def _chunk(c):
        off = base + c * CHUNK
        pltpu.sync_copy(ids_hbm.at[pl.ds(off, CHUNK)], idx_v)
        copies = [
            pltpu.async_copy(
                table_hbm.at[idx_v.at[pl.ds(j * G, G)]],
                rows_v.at[pl.ds(j * G, G)],
                sem,
            )
            for j in range(GPC)
        ]
        for cp in copies:
            cp.wait()
        pltpu.sync_copy(rows_v, out_hbm.at[pl.ds(off, CHUNK)])


_emb_call = functools.partial(
    pl.kernel,
    out_type=jax.ShapeDtypeStruct((B_TOTAL, DIM), jnp.float32),
    mesh=plsc.VectorSubcoreMesh(
        core_axis_name="c", subcore_axis_name="s", num_cores=NC, num_subcores=NS
    ),
    scratch_types=[
        pltpu.VMEM((CHUNK,), jnp.int32),
        pltpu.VMEM((CHUNK, DIM), jnp.float32),
        pltpu.SemaphoreType.DMA,
    ],
    compiler_params=pltpu.CompilerParams(use_tc_tiling_on_sc=False),
)(_emb_body)


@jax.jit
def kernel(ids, table):
    ids_flat = ids.reshape(-1).astype(jnp.int32)
    out = _emb_call(ids_flat, table)
    return out.reshape(ids.shape + (DIM,))


# speedup vs baseline: 1.1106x; 1.0011x over previous
import functools

import jax
import jax.numpy as jnp
from jax import lax
from jax.experimental import pallas as pl
from jax.experimental.pallas import tpu as pltpu
from jax.experimental.pallas import tpu_sc as plsc

NUM_EMB = 1_000_000
DIM = 32
B_TOTAL = 16384 * 50
NC, NS = 2, 16
NW = NC * NS
B_PER_W = B_TOTAL // NW
G = 128
CHUNK = 1280
GPC = CHUNK // G
NCHUNKS = B_PER_W // CHUNK
assert B_PER_W % CHUNK == 0 and CHUNK % G == 0 and NCHUNKS >= 2


def _emb_body(ids_hbm, table_hbm, out_hbm, idx_v, rows_v, sem_idx, sem_g, sem_out):
    wid = lax.axis_index("s") * NC + lax.axis_index("c")
    base = wid * B_PER_W

    def idx_copy(c, b):
        return pltpu.make_async_copy(
            ids_hbm.at[pl.ds(base + c * CHUNK, CHUNK)], idx_v.at[b], sem_idx
        )

    def out_copy(c, b):
        return pltpu.make_async_copy(
            rows_v.at[b], out_hbm.at[pl.ds(base + c * CHUNK, CHUNK)], sem_out
        )

    idx_copy(0, 0).start()

    @pl.loop(0, NCHUNKS)
    def _chunk(c):
        b = lax.rem(c, 2)
        idx_copy(c, b).wait()
        @pl.when(c >= 2)
        def _():
            out_copy(c - 2, b).wait()

        gathers = [
            pltpu.make_async_copy(
                table_hbm.at[idx_v.at[b].at[pl.ds(j * G, G)]],
                rows_v.at[b].at[pl.ds(j * G, G)],
                sem_g,
            )
            for j in range(GPC)
        ]
        for g in gathers:
            g.start()

        @pl.when(c + 1 < NCHUNKS)
        def _():
            idx_copy(c + 1, 1 - b).start()

        for g in gathers:
            g.wait()
        out_copy(c, b).start()

    out_copy(NCHUNKS - 2, (NCHUNKS - 2) % 2).wait()
    out_copy(NCHUNKS - 1, (NCHUNKS - 1) % 2).wait()


_emb_call = functools.partial(
    pl.kernel,
    out_type=jax.ShapeDtypeStruct((B_TOTAL, DIM), jnp.float32),
    mesh=plsc.VectorSubcoreMesh(
        core_axis_name="c", subcore_axis_name="s", num_cores=NC, num_subcores=NS
    ),
    scratch_types=[
        pltpu.VMEM((2, CHUNK), jnp.int32),
        pltpu.VMEM((2, CHUNK, DIM), jnp.float32),
        pltpu.SemaphoreType.DMA,
        pltpu.SemaphoreType.DMA,
        pltpu.SemaphoreType.DMA,
    ],
    compiler_params=pltpu.CompilerParams(use_tc_tiling_on_sc=False),
)(_emb_body)


@jax.jit
def kernel(ids, table):
    ids_flat = ids.reshape(-1).astype(jnp.int32)
    out = _emb_call(ids_flat, table)
    return out.reshape(ids.shape + (DIM,))


# speedup vs baseline: 1.7904x; 1.6122x over previous
import functools

import jax
import jax.numpy as jnp
from jax import lax
from jax.experimental import pallas as pl
from jax.experimental.pallas import tpu as pltpu
from jax.experimental.pallas import tpu_sc as plsc

NUM_EMB = 1_000_000
DIM = 32
BATCH = 16384
SEQ = 50
NC, NS = 2, 16
NW = NC * NS
ROWS_PER_W = BATCH // NW
BB = 16
NCHUNKS = ROWS_PER_W // BB
assert ROWS_PER_W % BB == 0 and NCHUNKS >= 2


def _emb_body(ids_hbm, table_hbm, out_hbm, idx_v, rows_v, sem_idx, sem_g, sem_out):
    wid = lax.axis_index("s") * NC + lax.axis_index("c")
    base = wid * ROWS_PER_W

    def idx_copy(c, b):
        return pltpu.make_async_copy(
            ids_hbm.at[pl.ds(base + c * BB, BB), :], idx_v.at[b], sem_idx
        )

    def out_copy(c, b):
        return pltpu.make_async_copy(
            rows_v.at[b], out_hbm.at[pl.ds(base + c * BB, BB), :, :], sem_out
        )

    idx_copy(0, 0).start()

    @pl.loop(0, NCHUNKS)
    def _chunk(c):
        b = lax.rem(c, 2)
        idx_copy(c, b).wait()
        @pl.when(c >= 2)
        def _():
            out_copy(c - 2, b).wait()

        gathers = [
            pltpu.make_async_copy(
                table_hbm.at[idx_v.at[b].at[i]],
                rows_v.at[b].at[i],
                sem_g,
            )
            for i in range(BB)
        ]
        for g in gathers:
            g.start()

        @pl.when(c + 1 < NCHUNKS)
        def _():
            idx_copy(c + 1, 1 - b).start()

        for g in gathers:
            g.wait()
        out_copy(c, b).start()

    out_copy(NCHUNKS - 2, (NCHUNKS - 2) % 2).wait()
    out_copy(NCHUNKS - 1, (NCHUNKS - 1) % 2).wait()


_emb_call = functools.partial(
    pl.kernel,
    out_type=jax.ShapeDtypeStruct((BATCH, SEQ, DIM), jnp.float32),
    mesh=plsc.VectorSubcoreMesh(
        core_axis_name="c", subcore_axis_name="s", num_cores=NC, num_subcores=NS
    ),
    scratch_types=[
        pltpu.VMEM((2, BB, SEQ), jnp.int32),
        pltpu.VMEM((2, BB, SEQ, DIM), jnp.float32),
        pltpu.SemaphoreType.DMA,
        pltpu.SemaphoreType.DMA,
        pltpu.SemaphoreType.DMA,
    ],
    compiler_params=pltpu.CompilerParams(use_tc_tiling_on_sc=False),
)(_emb_body)


@jax.jit
def kernel(ids, table):
    return _emb_call(ids.astype(jnp.int32), table)
